# Initial kernel scaffold; baseline (speedup 1.0000x reference)
import jax
import jax.numpy as jnp
from jax.experimental import pallas as pl


def kernel(x, y, y_pos, W_cluster, logits):
    raise NotImplementedError("write your pallas kernel here")


# speedup vs baseline: 1.7503x; 1.7503x over previous
import functools

import jax
import jax.numpy as jnp
from jax import lax
from jax.experimental import pallas as pl
from jax.experimental.pallas import tpu as pltpu

VOCAB = 16384
HIDDEN = 1024
N_CLUSTERS = 16
S = VOCAB // N_CLUSTERS
N_TOKENS = 4096
BLK = 128
NBLK = N_TOKENS // BLK


def _fused_body(y_ref, ypos_ref, x_ref, w_ref, wc_ref, out_ref,
                acc_ref, xb_ref, wb_ref, wcb_ref):
    c = pl.program_id(0)
    i = pl.program_id(1)

    @pl.when(jnp.logical_and(c == 0, i == 0))
    def _():
        xb_ref[...] = x_ref[...].astype(jnp.bfloat16)
        wcb_ref[...] = wc_ref[...].astype(jnp.bfloat16)

    @pl.when(i == 0)
    def _():
        wb_ref[...] = w_ref[...].astype(jnp.bfloat16)

    xblk = xb_ref[pl.ds(i * BLK, BLK), :]
    ypos = ypos_ref[0]

    @pl.when(c == 0)
    def _():
        cl = lax.dot_general(xblk, wcb_ref[...], (((1,), (1,)), ((), ())),
                             preferred_element_type=jnp.float32)
        m = jnp.max(cl, axis=1, keepdims=True)
        lse = jnp.log(jnp.sum(jnp.exp(cl - m), axis=1, keepdims=True)) + m
        sel = jnp.sum(jnp.where(
            lax.broadcasted_iota(jnp.int32, (BLK, N_CLUSTERS), 1) == ypos,
            cl, 0.0), axis=1, keepdims=True)
        acc_ref[i] = lse - sel

    t = jnp.dot(xblk, wb_ref[...],
                preferred_element_type=jnp.float32)
    m = jnp.max(t, axis=1, keepdims=True)
    lse = jnp.log(jnp.sum(jnp.exp(t - m), axis=1, keepdims=True)) + m
    tcol = y_ref[0] & (S - 1)
    tgt = jnp.sum(jnp.where(
        lax.broadcasted_iota(jnp.int32, (BLK, S), 1) == tcol,
        t, 0.0), axis=1, keepdims=True)
    hit = ypos == c
    acc_ref[i] = acc_ref[i] + jnp.where(hit, lse - tgt, 0.0)
    out_ref[0] = acc_ref[i]


def kernel(x, y, y_pos, W_cluster, logits):
    y3 = y.reshape(NBLK, BLK, 1)
    ypos3 = y_pos.reshape(NBLK, BLK, 1)
    out = pl.pallas_call(
        _fused_body,
        grid=(N_CLUSTERS, NBLK),
        in_specs=[
            pl.BlockSpec((1, BLK, 1), lambda c, i: (i, 0, 0)),
            pl.BlockSpec((1, BLK, 1), lambda c, i: (i, 0, 0)),
            pl.BlockSpec((N_TOKENS, HIDDEN), lambda c, i: (0, 0)),
            pl.BlockSpec((HIDDEN, S), lambda c, i: (0, c)),
            pl.BlockSpec((N_CLUSTERS, HIDDEN), lambda c, i: (0, 0)),
        ],
        out_specs=pl.BlockSpec((1, BLK, 1), lambda c, i: (i, 0, 0)),
        out_shape=jax.ShapeDtypeStruct((NBLK, BLK, 1), jnp.float32),
        scratch_shapes=[
            pltpu.VMEM((NBLK, BLK, 1), jnp.float32),
            pltpu.VMEM((N_TOKENS, HIDDEN), jnp.bfloat16),
            pltpu.VMEM((HIDDEN, S), jnp.bfloat16),
            pltpu.VMEM((N_CLUSTERS, HIDDEN), jnp.bfloat16),
        ],
    )(y3, ypos3, x, logits, W_cluster)
    return out.reshape(N_TOKENS)


# speedup vs baseline: 3.2133x; 1.8359x over previous
import jax
import jax.numpy as jnp
from jax import lax
from jax.experimental import pallas as pl
from jax.experimental.pallas import tpu as pltpu
from jax.experimental.pallas import tpu_sc as plsc

VOCAB = 16384
HIDDEN = 1024
C = 16
S = VOCAB // C
N = 4096
B = 128
NBLK = N // B + C
NPAD = NBLK * B
L = 16
NSUB = 16
TPW = N // NSUB
SPAD = 8
GW = 32
RPW = NPAD // GW
RCH = RPW // 2


def _lane():
    return lax.broadcasted_iota(jnp.int32, (L,), 0)


def _route_body(ypos_hbm, y_hbm, slot_hbm, perm_hbm, tcol_hbm, bc_hbm,
                yp_v, y_v, hist_v, all_v, slot_v,
                sidx0_v, sidx1_v, tok0_v, tok1_v, tc0_v, tc1_v, bc_v,
                shared_hist, sem):
    w = lax.axis_index("s")
    t0 = w * TPW
    lane = _lane()
    pltpu.sync_copy(ypos_hbm.at[pl.ds(t0, TPW)], yp_v)
    pltpu.sync_copy(y_hbm.at[pl.ds(t0, TPW)], y_v)

    hist = jnp.zeros((L,), jnp.int32)
    for k in range(TPW // L):
        v = yp_v[pl.ds(k * L, L)]
        for b in range(C):
            cnt = jnp.sum(jnp.where(v == b, 1, 0))
            hist = hist + jnp.where(lane == b, cnt, 0)
    hist_v[...] = hist
    pltpu.sync_copy(hist_v, shared_hist.at[w + SPAD])
    plsc.subcore_barrier()
    plsc.subcore_barrier()
    pltpu.sync_copy(shared_hist.at[pl.ds(SPAD, NSUB)], all_v)

    prefix = jnp.zeros((L,), jnp.int32)
    total = jnp.zeros((L,), jnp.int32)
    for w2 in range(NSUB):
        h = all_v[w2]
        prefix = prefix + h * jnp.where(w2 < w, 1, 0)
        total = total + h
    nblk = (total + (B - 1)) >> 7
    blkstart = plsc.cumsum(nblk) - nblk
    base = (blkstart << 7) + prefix

    cur = base
    for k in range(TPW // L):
        v = yp_v[pl.ds(k * L, L)]
        yv = y_v[pl.ds(k * L, L)]
        slotv = jnp.zeros((L,), jnp.int32)
        newcur = cur
        for b in range(C):
            m = v == b
            mi = jnp.where(m, 1, 0)
            rank = plsc.cumsum(mi) - 1
            cur_b = jnp.sum(jnp.where(lane == b, cur, 0))
            slotv = jnp.where(m, cur_b + rank, slotv)
            newcur = newcur + jnp.where(lane == b, jnp.sum(mi), 0)
        cur = newcur
        slot_v[pl.ds(k * L, L)] = slotv
        tokv = t0 + k * L + lane
        tcv = yv & (S - 1)
        o = (k % 8) * L
        if k < 8:
            sidx0_v[pl.ds(o, L)] = slotv
            tok0_v[pl.ds(o, L)] = tokv
            tc0_v[pl.ds(o, L)] = tcv
        else:
            sidx1_v[pl.ds(o, L)] = slotv
            tok1_v[pl.ds(o, L)] = tokv
            tc1_v[pl.ds(o, L)] = tcv

    pltpu.sync_copy(slot_v, slot_hbm.at[pl.ds(t0, TPW)])
    d0 = pltpu.async_copy(tok0_v, perm_hbm.at[sidx0_v], sem)
    d1 = pltpu.async_copy(tok1_v, perm_hbm.at[sidx1_v], sem)
    d2 = pltpu.async_copy(tc0_v, tcol_hbm.at[sidx0_v], sem)
    d3 = pltpu.async_copy(tc1_v, tcol_hbm.at[sidx1_v], sem)
    d0.wait()
    d1.wait()
    d2.wait()
    d3.wait()

    @pl.when(w == 0)
    def _():
        hc = jnp.max(jnp.where(total > 0, lane, 0))
        for r in range(NBLK // L):
            bi = lane + r * L
            acc = jnp.zeros((L,), jnp.int32) + hc
            for b in range(C):
                st = jnp.sum(jnp.where(lane == b, blkstart, 0))
                nb = jnp.sum(jnp.where(lane == b, nblk, 0))
                acc = jnp.where((bi >= st) & (bi < st + nb), b, acc)
            bc_v[pl.ds(r * L, L)] = acc
        pltpu.sync_copy(bc_v, bc_hbm)


def _route(y_pos, y):
    mesh = plsc.VectorSubcoreMesh(
        core_axis_name="c", subcore_axis_name="s", num_cores=1)
    f = pl.kernel(
        _route_body,
        compiler_params=pltpu.CompilerParams(needs_layout_passes=False),
        out_type=(
            jax.ShapeDtypeStruct((N,), jnp.int32),
            jax.ShapeDtypeStruct((NPAD,), jnp.int32),
            jax.ShapeDtypeStruct((NPAD,), jnp.int32),
            jax.ShapeDtypeStruct((NBLK,), jnp.int32),
        ),
        mesh=mesh,
        scratch_types=[
            pltpu.VMEM((TPW,), jnp.int32),
            pltpu.VMEM((TPW,), jnp.int32),
            pltpu.VMEM((L,), jnp.int32),
            pltpu.VMEM((NSUB, L), jnp.int32),
            pltpu.VMEM((TPW,), jnp.int32),
            pltpu.VMEM((128,), jnp.int32),
            pltpu.VMEM((128,), jnp.int32),
            pltpu.VMEM((128,), jnp.int32),
            pltpu.VMEM((128,), jnp.int32),
            pltpu.VMEM((128,), jnp.int32),
            pltpu.VMEM((128,), jnp.int32),
            pltpu.VMEM((NBLK,), jnp.int32),
            pltpu.VMEM_SHARED((NSUB + SPAD, L), jnp.int32),
            pltpu.SemaphoreType.DMA,
        ],
    )
    return f(y_pos, y)


def _gather_body(x_hbm, perm_hbm, xs_hbm, idx_v, rows_v, sem):
    wid = lax.axis_index("s") * 2 + lax.axis_index("c")
    base = wid * RPW
    for j in range(2):
        pltpu.sync_copy(perm_hbm.at[pl.ds(base + j * RCH, RCH)], idx_v)
        for q in range(RCH // L):
            iv = idx_v[pl.ds(q * L, L)]
            idx_v[pl.ds(q * L, L)] = jnp.minimum(jnp.maximum(iv, 0), N - 1)
        pltpu.async_copy(x_hbm.at[idx_v], rows_v, sem).wait()
        pltpu.sync_copy(rows_v, xs_hbm.at[pl.ds(base + j * RCH, RCH)])


def _gather_rows(x, perm):
    mesh = plsc.VectorSubcoreMesh(core_axis_name="c", subcore_axis_name="s")
    f = pl.kernel(
        _gather_body,
        out_type=jax.ShapeDtypeStruct((NPAD, HIDDEN), jnp.float32),
        mesh=mesh,
        scratch_types=[
            pltpu.VMEM((RCH,), jnp.int32),
            pltpu.VMEM((RCH, HIDDEN), jnp.float32),
            pltpu.SemaphoreType.DMA,
        ],
    )
    return f(x, perm)


def _mm_body(bc_ref, xs_ref, w_ref, wc_ref, tcol_ref, out_ref, wb_ref):
    i = pl.program_id(0)
    c = bc_ref[i]
    prev = bc_ref[jnp.maximum(i - 1, 0)]

    @pl.when((i == 0) | (c != prev))
    def _():
        wb_ref[...] = w_ref[...].astype(jnp.bfloat16)

    xb = xs_ref[...].astype(jnp.bfloat16)
    t = jnp.dot(xb, wb_ref[...],
                preferred_element_type=jnp.float32)
    m = jnp.max(t, axis=1, keepdims=True)
    lse = jnp.log(jnp.sum(jnp.exp(t - m), axis=1, keepdims=True)) + m
    tcol = tcol_ref[0]
    tgt = jnp.sum(jnp.where(
        lax.broadcasted_iota(jnp.int32, (B, S), 1) == tcol, t, 0.0),
        axis=1, keepdims=True)

    cl = lax.dot_general(xb, wc_ref[...].astype(jnp.bfloat16),
                         (((1,), (1,)), ((), ())),
                         preferred_element_type=jnp.float32)
    mc = jnp.max(cl, axis=1, keepdims=True)
    lse_c = jnp.log(jnp.sum(jnp.exp(cl - mc), axis=1, keepdims=True)) + mc
    sel = jnp.sum(jnp.where(
        lax.broadcasted_iota(jnp.int32, (B, C), 1) == c, cl, 0.0),
        axis=1, keepdims=True)

    out_ref[0] = (lse_c - sel) + (lse - tgt)


def _tail_nll(bc, xs, logits, W_cluster, tcol):
    grid_spec = pltpu.PrefetchScalarGridSpec(
        num_scalar_prefetch=1,
        grid=(NBLK,),
        in_specs=[
            pl.BlockSpec((B, HIDDEN), lambda i, bc: (i, 0)),
            pl.BlockSpec((HIDDEN, S), lambda i, bc: (0, bc[i])),
            pl.BlockSpec((C, HIDDEN), lambda i, bc: (0, 0)),
            pl.BlockSpec((1, B, 1), lambda i, bc: (i, 0, 0)),
        ],
        out_specs=pl.BlockSpec((1, B, 1), lambda i, bc: (i, 0, 0)),
        scratch_shapes=[pltpu.VMEM((HIDDEN, S), jnp.bfloat16)],
    )
    return pl.pallas_call(
        _mm_body,
        grid_spec=grid_spec,
        out_shape=jax.ShapeDtypeStruct((NBLK, B, 1), jnp.float32),
    )(bc, xs, logits, W_cluster, tcol.reshape(NBLK, B, 1))


def _final_body(slot_hbm, nlls_hbm, out_hbm, sidx_v, vals_v, sem):
    wid = lax.axis_index("s") * 2 + lax.axis_index("c")
    base = wid * (N // GW)
    pltpu.sync_copy(slot_hbm.at[pl.ds(base, N // GW)], sidx_v)
    for q in range((N // GW) // L):
        iv = sidx_v[pl.ds(q * L, L)]
        sidx_v[pl.ds(q * L, L)] = jnp.minimum(jnp.maximum(iv, 0), NPAD - 1)
    pltpu.async_copy(nlls_hbm.at[sidx_v], vals_v, sem).wait()
    pltpu.sync_copy(vals_v, out_hbm.at[pl.ds(base, N // GW)])


def _finalize(slot, nlls):
    mesh = plsc.VectorSubcoreMesh(core_axis_name="c", subcore_axis_name="s")
    f = pl.kernel(
        _final_body,
        out_type=jax.ShapeDtypeStruct((N,), jnp.float32),
        mesh=mesh,
        scratch_types=[
            pltpu.VMEM((N // GW,), jnp.int32),
            pltpu.VMEM((N // GW,), jnp.float32),
            pltpu.SemaphoreType.DMA,
        ],
    )
    return f(slot, nlls)


def kernel(x, y, y_pos, W_cluster, logits):
    slot, perm, tcol, bc = _route(y_pos, y)
    xs = _gather_rows(x, perm)
    nlls = _tail_nll(bc, xs, logits, W_cluster, tcol)
    return _finalize(slot, nlls.reshape(NPAD))


# speedup vs baseline: 3.2437x; 1.0095x over previous
import jax
import jax.numpy as jnp
from jax import lax
from jax.experimental import pallas as pl
from jax.experimental.pallas import tpu as pltpu
from jax.experimental.pallas import tpu_sc as plsc

VOCAB = 16384
HIDDEN = 1024
C = 16
S = VOCAB // C
N = 4096
B = 128
NBLK = N // B + C
NPAD = NBLK * B
L = 16
NSUB = 16
TPW = N // NSUB
SPAD = 8
GW = 32
RPW = NPAD // GW
RCH = RPW // 2


def _lane():
    return lax.broadcasted_iota(jnp.int32, (L,), 0)


_GDN = lax.GatherDimensionNumbers(
    offset_dims=(), collapsed_slice_dims=(0,), start_index_map=(0,))


def _vperm(vals, idx):
    return lax.gather(vals, idx[:, None], _GDN, (1,),
                      mode=lax.GatherScatterMode.PROMISE_IN_BOUNDS)


def _route_body(ypos_hbm, y_hbm, slot_hbm, perm_hbm, tcol_hbm, bc_hbm,
                yp_v, y_v, hist_v, cur_v, all_v, slot_v,
                sidx0_v, sidx1_v, tok0_v, tok1_v, tc0_v, tc1_v, bc_v,
                shared_hist, sem):
    w = lax.axis_index("s")
    t0 = w * TPW
    lane = _lane()
    pltpu.sync_copy(ypos_hbm.at[pl.ds(t0, TPW)], yp_v)
    pltpu.sync_copy(y_hbm.at[pl.ds(t0, TPW)], y_v)

    ones = jnp.ones((L,), jnp.int32)
    hist_v[...] = jnp.zeros((L,), jnp.int32)
    for k in range(TPW // L):
        v = yp_v[pl.ds(k * L, L)]
        plsc.addupdate_scatter(hist_v, [v], ones)
    pltpu.sync_copy(hist_v, shared_hist.at[w + SPAD])
    plsc.subcore_barrier()
    plsc.subcore_barrier()
    pltpu.sync_copy(shared_hist.at[pl.ds(SPAD, NSUB)], all_v)

    prefix = jnp.zeros((L,), jnp.int32)
    total = jnp.zeros((L,), jnp.int32)
    for w2 in range(NSUB):
        h = all_v[w2]
        prefix = prefix + h * jnp.where(w2 < w, 1, 0)
        total = total + h
    nblk = (total + (B - 1)) >> 7
    blkstart = plsc.cumsum(nblk) - nblk
    base = (blkstart << 7) + prefix

    cur_v[...] = base
    for k in range(TPW // L):
        v = yp_v[pl.ds(k * L, L)]
        yv = y_v[pl.ds(k * L, L)]
        rank = jnp.zeros((L,), jnp.int32)
        for sft in range(1, L):
            sh = _vperm(v, jnp.maximum(lane - sft, 0))
            rank = rank + jnp.where((lane >= sft) & (sh == v), 1, 0)
        slotv = _vperm(cur_v[...], v) + rank
        plsc.addupdate_scatter(cur_v, [v], ones)
        slot_v[pl.ds(k * L, L)] = slotv
        tokv = t0 + k * L + lane
        tcv = yv & (S - 1)
        o = (k % 8) * L
        if k < 8:
            sidx0_v[pl.ds(o, L)] = slotv
            tok0_v[pl.ds(o, L)] = tokv
            tc0_v[pl.ds(o, L)] = tcv
        else:
            sidx1_v[pl.ds(o, L)] = slotv
            tok1_v[pl.ds(o, L)] = tokv
            tc1_v[pl.ds(o, L)] = tcv

    pltpu.sync_copy(slot_v, slot_hbm.at[pl.ds(t0, TPW)])
    d0 = pltpu.async_copy(tok0_v, perm_hbm.at[sidx0_v], sem)
    d1 = pltpu.async_copy(tok1_v, perm_hbm.at[sidx1_v], sem)
    d2 = pltpu.async_copy(tc0_v, tcol_hbm.at[sidx0_v], sem)
    d3 = pltpu.async_copy(tc1_v, tcol_hbm.at[sidx1_v], sem)
    d0.wait()
    d1.wait()
    d2.wait()
    d3.wait()

    @pl.when(w == 0)
    def _():
        for r in range(NBLK // L):
            bi = lane + r * L
            cnt = jnp.zeros((L,), jnp.int32)
            for b in range(C):
                st = _vperm(blkstart, jnp.zeros((L,), jnp.int32) + b)
                cnt = cnt + jnp.where(st <= bi, 1, 0)
            bc_v[pl.ds(r * L, L)] = jnp.minimum(cnt - 1, C - 1)
        pltpu.sync_copy(bc_v, bc_hbm)


def _route(y_pos, y):
    mesh = plsc.VectorSubcoreMesh(
        core_axis_name="c", subcore_axis_name="s", num_cores=1)
    f = pl.kernel(
        _route_body,
        compiler_params=pltpu.CompilerParams(needs_layout_passes=False),
        out_type=(
            jax.ShapeDtypeStruct((N,), jnp.int32),
            jax.ShapeDtypeStruct((NPAD,), jnp.int32),
            jax.ShapeDtypeStruct((NPAD,), jnp.int32),
            jax.ShapeDtypeStruct((NBLK,), jnp.int32),
        ),
        mesh=mesh,
        scratch_types=[
            pltpu.VMEM((TPW,), jnp.int32),
            pltpu.VMEM((TPW,), jnp.int32),
            pltpu.VMEM((L,), jnp.int32),
            pltpu.VMEM((L,), jnp.int32),
            pltpu.VMEM((NSUB, L), jnp.int32),
            pltpu.VMEM((TPW,), jnp.int32),
            pltpu.VMEM((128,), jnp.int32),
            pltpu.VMEM((128,), jnp.int32),
            pltpu.VMEM((128,), jnp.int32),
            pltpu.VMEM((128,), jnp.int32),
            pltpu.VMEM((128,), jnp.int32),
            pltpu.VMEM((128,), jnp.int32),
            pltpu.VMEM((NBLK,), jnp.int32),
            pltpu.VMEM_SHARED((NSUB + SPAD, L), jnp.int32),
            pltpu.SemaphoreType.DMA,
        ],
    )
    return f(y_pos, y)


NCH = 4
GCH = RPW // NCH


def _gather_body(x_hbm, perm_hbm, xs_hbm, idx_v, rows0_v, rows1_v,
                 sem0, sem1):
    wid = lax.axis_index("s") * 2 + lax.axis_index("c")
    base = wid * RPW
    for j in range(NCH):
        pltpu.sync_copy(perm_hbm.at[pl.ds(base + j * GCH, GCH)], idx_v.at[j])
    for q in range(RPW // L):
        r, o = q // (GCH // L), (q % (GCH // L)) * L
        iv = idx_v[r, pl.ds(o, L)]
        idx_v[r, pl.ds(o, L)] = jnp.minimum(jnp.maximum(iv, 0), N - 1)
    rows = (rows0_v, rows1_v)
    sems = (sem0, sem1)
    pend = [None, None]
    pend[0] = pltpu.async_copy(x_hbm.at[idx_v.at[0]], rows0_v, sem0)
    for j in range(NCH):
        pend[j % 2].wait()
        if j + 1 < NCH:
            pend[(j + 1) % 2] = pltpu.async_copy(
                x_hbm.at[idx_v.at[j + 1]], rows[(j + 1) % 2],
                sems[(j + 1) % 2])
        pltpu.sync_copy(rows[j % 2], xs_hbm.at[pl.ds(base + j * GCH, GCH)])


def _gather_rows(x, perm):
    mesh = plsc.VectorSubcoreMesh(core_axis_name="c", subcore_axis_name="s")
    f = pl.kernel(
        _gather_body,
        out_type=jax.ShapeDtypeStruct((NPAD, HIDDEN), jnp.float32),
        mesh=mesh,
        scratch_types=[
            pltpu.VMEM((NCH, GCH), jnp.int32),
            pltpu.VMEM((GCH, HIDDEN), jnp.float32),
            pltpu.VMEM((GCH, HIDDEN), jnp.float32),
            pltpu.SemaphoreType.DMA,
            pltpu.SemaphoreType.DMA,
        ],
    )
    return f(x, perm)


def _mm_body(bc_ref, xs_ref, w_ref, wc_ref, tcol_ref, out_ref, wb_ref):
    i = pl.program_id(0)
    c = bc_ref[i]
    prev = bc_ref[jnp.maximum(i - 1, 0)]

    @pl.when((i == 0) | (c != prev))
    def _():
        wb_ref[...] = w_ref[...].astype(jnp.bfloat16)

    xb = xs_ref[...].astype(jnp.bfloat16)
    t = jnp.dot(xb, wb_ref[...],
                preferred_element_type=jnp.float32)
    m = jnp.max(t, axis=1, keepdims=True)
    lse = jnp.log(jnp.sum(jnp.exp(t - m), axis=1, keepdims=True)) + m
    tcol = tcol_ref[0]
    tgt = jnp.sum(jnp.where(
        lax.broadcasted_iota(jnp.int32, (B, S), 1) == tcol, t, 0.0),
        axis=1, keepdims=True)

    cl = lax.dot_general(xb, wc_ref[...].astype(jnp.bfloat16),
                         (((1,), (1,)), ((), ())),
                         preferred_element_type=jnp.float32)
    mc = jnp.max(cl, axis=1, keepdims=True)
    lse_c = jnp.log(jnp.sum(jnp.exp(cl - mc), axis=1, keepdims=True)) + mc
    sel = jnp.sum(jnp.where(
        lax.broadcasted_iota(jnp.int32, (B, C), 1) == c, cl, 0.0),
        axis=1, keepdims=True)

    out_ref[0] = (lse_c - sel) + (lse - tgt)


def _tail_nll(bc, xs, logits, W_cluster, tcol):
    grid_spec = pltpu.PrefetchScalarGridSpec(
        num_scalar_prefetch=1,
        grid=(NBLK,),
        in_specs=[
            pl.BlockSpec((B, HIDDEN), lambda i, bc: (i, 0)),
            pl.BlockSpec((HIDDEN, S), lambda i, bc: (0, bc[i])),
            pl.BlockSpec((C, HIDDEN), lambda i, bc: (0, 0)),
            pl.BlockSpec((1, B, 1), lambda i, bc: (i, 0, 0)),
        ],
        out_specs=pl.BlockSpec((1, B, 1), lambda i, bc: (i, 0, 0)),
        scratch_shapes=[pltpu.VMEM((HIDDEN, S), jnp.bfloat16)],
    )
    return pl.pallas_call(
        _mm_body,
        grid_spec=grid_spec,
        out_shape=jax.ShapeDtypeStruct((NBLK, B, 1), jnp.float32),
    )(bc, xs, logits, W_cluster, tcol.reshape(NBLK, B, 1))


def _final_body(slot_hbm, nlls_hbm, out_hbm, sidx_v, vals_v, sem):
    wid = lax.axis_index("s") * 2 + lax.axis_index("c")
    base = wid * (N // GW)
    pltpu.sync_copy(slot_hbm.at[pl.ds(base, N // GW)], sidx_v)
    for q in range((N // GW) // L):
        iv = sidx_v[pl.ds(q * L, L)]
        sidx_v[pl.ds(q * L, L)] = jnp.minimum(jnp.maximum(iv, 0), NPAD - 1)
    pltpu.async_copy(nlls_hbm.at[sidx_v], vals_v, sem).wait()
    pltpu.sync_copy(vals_v, out_hbm.at[pl.ds(base, N // GW)])


def _finalize(slot, nlls):
    mesh = plsc.VectorSubcoreMesh(core_axis_name="c", subcore_axis_name="s")
    f = pl.kernel(
        _final_body,
        out_type=jax.ShapeDtypeStruct((N,), jnp.float32),
        mesh=mesh,
        scratch_types=[
            pltpu.VMEM((N // GW,), jnp.int32),
            pltpu.VMEM((N // GW,), jnp.float32),
            pltpu.SemaphoreType.DMA,
        ],
    )
    return f(slot, nlls)


def kernel(x, y, y_pos, W_cluster, logits):
    slot, perm, tcol, bc = _route(y_pos, y)
    xs = _gather_rows(x, perm)
    nlls = _tail_nll(bc, xs, logits, W_cluster, tcol)
    return _finalize(slot, nlls.reshape(NPAD))
